# Initial kernel scaffold; baseline (speedup 1.0000x reference)
import jax
import jax.numpy as jnp
from jax.experimental import pallas as pl


def kernel(embedding_table):
    raise NotImplementedError("write your pallas kernel here")


# speedup vs baseline: 3.4812x; 3.4812x over previous
import jax
import jax.numpy as jnp
from jax.experimental import pallas as pl

NUM_NODES = 100000
EMBED_DIM = 128
BLOCK_ROWS = 10000


def _copy_body(in_ref, out_ref):
    out_ref[...] = in_ref[...]


def kernel(embedding_table):
    n, d = embedding_table.shape
    grid = (n // BLOCK_ROWS,)
    return pl.pallas_call(
        _copy_body,
        grid=grid,
        in_specs=[pl.BlockSpec((BLOCK_ROWS, d), lambda i: (i, 0))],
        out_specs=pl.BlockSpec((BLOCK_ROWS, d), lambda i: (i, 0)),
        out_shape=jax.ShapeDtypeStruct((n, d), embedding_table.dtype),
    )(embedding_table)


# speedup vs baseline: 3.6006x; 1.0343x over previous
import jax
import jax.numpy as jnp
from jax.experimental import pallas as pl

NUM_NODES = 100000
EMBED_DIM = 128
BLOCK_ROWS = 20000


def _copy_body(in_ref, out_ref):
    out_ref[...] = in_ref[...]


def kernel(embedding_table):
    n, d = embedding_table.shape
    grid = (n // BLOCK_ROWS,)
    return pl.pallas_call(
        _copy_body,
        grid=grid,
        in_specs=[pl.BlockSpec((BLOCK_ROWS, d), lambda i: (i, 0))],
        out_specs=pl.BlockSpec((BLOCK_ROWS, d), lambda i: (i, 0)),
        out_shape=jax.ShapeDtypeStruct((n, d), embedding_table.dtype),
    )(embedding_table)
